# baseline (speedup 1.0000x reference)
import functools

import jax
import jax.numpy as jnp
from jax import lax
from jax.experimental import pallas as pl
from jax.experimental.pallas import tpu as pltpu, tpu_sc as plsc

NUM_CLASSES = 100000
EMB_DIM = 64
BATCH = 16384

_NW = 32
_B_PER_W = BATCH // _NW
_CHUNK = 128
_NCHUNKS = _B_PER_W // _CHUNK


def _make_gather():
    mesh = plsc.VectorSubcoreMesh(core_axis_name="c", subcore_axis_name="s")

    @functools.partial(
        pl.kernel,
        mesh=mesh,
        out_type=jax.ShapeDtypeStruct((BATCH, 128), jnp.float32),
        scratch_types=[
            pltpu.VMEM((_NCHUNKS, _CHUNK), jnp.int32),
            pltpu.VMEM((_B_PER_W, 128), jnp.float32),
            pltpu.SemaphoreType.DMA((_NCHUNKS,)),
            pltpu.SemaphoreType.DMA,
        ],
    )
    def gather_kernel(idx_hbm, tpad_hbm, out_hbm, idx_v, rows_v, sem, sem_w):
        wid = lax.axis_index("s") * 2 + lax.axis_index("c")
        base = wid * _B_PER_W
        pltpu.sync_copy(idx_hbm.at[wid], idx_v)
        gathers = []
        writes = []
        for j in range(_NCHUNKS):
            gathers.append(
                pltpu.make_async_copy(
                    tpad_hbm.at[idx_v.at[j]],
                    rows_v.at[pl.ds(j * _CHUNK, _CHUNK)],
                    sem.at[j],
                )
            )
            gathers[-1].start()
        for j in range(_NCHUNKS):
            gathers[j].wait()
            writes.append(
                pltpu.make_async_copy(
                    rows_v.at[pl.ds(j * _CHUNK, _CHUNK)],
                    out_hbm.at[pl.ds(base + j * _CHUNK, _CHUNK)],
                    sem_w,
                )
            )
            writes[-1].start()
        for w in writes:
            w.wait()

    return gather_kernel


_gather = _make_gather()

_TBLK = 512
_TGRID = (NUM_CLASSES + _TBLK - 1) // _TBLK


def _xpose_pad_kernel(t2_ref, out_ref):
    out_ref[:, :EMB_DIM] = t2_ref[...].T
    out_ref[:, EMB_DIM:] = jnp.zeros((_TBLK, 128 - EMB_DIM), jnp.float32)


_xpose_pad = pl.pallas_call(
    _xpose_pad_kernel,
    grid=(_TGRID,),
    in_specs=[pl.BlockSpec((EMB_DIM, _TBLK), lambda i: (0, i))],
    out_specs=pl.BlockSpec((_TBLK, 128), lambda i: (i, 0)),
    out_shape=jax.ShapeDtypeStruct((NUM_CLASSES, 128), jnp.float32),
)


@jax.jit
def kernel(class_indices, table):
    tpad = _xpose_pad(table.T)
    idx = class_indices.reshape(_NW, _NCHUNKS, _CHUNK)
    padded = _gather(idx, tpad)
    return padded[:, :EMB_DIM]


# speedup vs baseline: 1.7784x; 1.7784x over previous
import functools

import jax
import jax.numpy as jnp
from jax import lax
from jax.experimental import pallas as pl
from jax.experimental.pallas import tpu as pltpu, tpu_sc as plsc

NUM_CLASSES = 100000
EMB_DIM = 64
BATCH = 16384

_NW = 32
_B_PER_W = BATCH // _NW
_CHUNK = 128
_NCHUNKS = _B_PER_W // _CHUNK


def _make_gather():
    mesh = plsc.VectorSubcoreMesh(core_axis_name="c", subcore_axis_name="s")

    @functools.partial(
        pl.kernel,
        mesh=mesh,
        out_type=jax.ShapeDtypeStruct((BATCH, 128), jnp.float32),
        scratch_types=[
            pltpu.VMEM((_NCHUNKS, _CHUNK), jnp.int32),
            pltpu.VMEM((_B_PER_W, 128), jnp.float32),
            pltpu.SemaphoreType.DMA((_NCHUNKS,)),
            pltpu.SemaphoreType.DMA,
        ],
    )
    def gather_kernel(idx_hbm, tpad_hbm, out_hbm, idx_v, rows_v, sem, sem_w):
        wid = lax.axis_index("s") * 2 + lax.axis_index("c")
        base = wid * _B_PER_W
        pltpu.sync_copy(idx_hbm.at[wid], idx_v)
        gathers = []
        writes = []
        for j in range(_NCHUNKS):
            gathers.append(
                pltpu.make_async_copy(
                    tpad_hbm.at[idx_v.at[j]],
                    rows_v.at[pl.ds(j * _CHUNK, _CHUNK)],
                    sem.at[j],
                )
            )
            gathers[-1].start()
        for j in range(_NCHUNKS):
            gathers[j].wait()
            writes.append(
                pltpu.make_async_copy(
                    rows_v.at[pl.ds(j * _CHUNK, _CHUNK)],
                    out_hbm.at[pl.ds(base + j * _CHUNK, _CHUNK)],
                    sem_w,
                )
            )
            writes[-1].start()
        for w in writes:
            w.wait()

    return gather_kernel


_gather = _make_gather()


@jax.jit
def kernel(class_indices, table):
    tpad = jnp.pad(table, ((0, 0), (0, 128 - EMB_DIM)))
    idx = class_indices.reshape(_NW, _NCHUNKS, _CHUNK)
    padded = _gather(idx, tpad)
    return padded[:, :EMB_DIM]


# speedup vs baseline: 1.9418x; 1.0919x over previous
import functools

import jax
import jax.numpy as jnp
from jax import lax
from jax.experimental import pallas as pl
from jax.experimental.pallas import tpu as pltpu, tpu_sc as plsc

NUM_CLASSES = 100000
EMB_DIM = 64
BATCH = 16384

_NW = 32
_B_PER_W = BATCH // _NW
_CHUNK = 128
_NCHUNKS = _B_PER_W // _CHUNK


def _make_gather():
    mesh = plsc.VectorSubcoreMesh(core_axis_name="c", subcore_axis_name="s")

    @functools.partial(
        pl.kernel,
        mesh=mesh,
        out_type=jax.ShapeDtypeStruct((BATCH, 128), jnp.float32),
        scratch_types=[
            pltpu.VMEM((_NCHUNKS, _CHUNK), jnp.int32),
            pltpu.VMEM((_B_PER_W, 128), jnp.float32),
            pltpu.SemaphoreType.DMA((_NCHUNKS,)),
            pltpu.SemaphoreType.DMA,
        ],
    )
    def gather_kernel(idx_hbm, tpad_hbm, out_hbm, idx_v, rows_v, sem, sem_w):
        wid = lax.axis_index("s") * 2 + lax.axis_index("c")
        base = wid * _B_PER_W
        pltpu.sync_copy(idx_hbm.at[wid], idx_v)
        gathers = []
        writes = []
        for j in range(_NCHUNKS):
            gathers.append(
                pltpu.make_async_copy(
                    tpad_hbm.at[idx_v.at[j]],
                    rows_v.at[pl.ds(j * _CHUNK, _CHUNK)],
                    sem.at[j],
                )
            )
            gathers[-1].start()
        for j in range(_NCHUNKS):
            gathers[j].wait()
            writes.append(
                pltpu.make_async_copy(
                    rows_v.at[pl.ds(j * _CHUNK, _CHUNK)],
                    out_hbm.at[pl.ds(base + j * _CHUNK, _CHUNK)],
                    sem_w,
                )
            )
            writes[-1].start()
        for w in writes:
            w.wait()

    return gather_kernel


_gather = _make_gather()

_TBLK = 2048
_TGRID = (NUM_CLASSES + _TBLK - 1) // _TBLK


def _xpose_pad_kernel(t2_ref, out_ref):
    out_ref[:, :EMB_DIM] = t2_ref[...].T
    out_ref[:, EMB_DIM:] = jnp.zeros((_TBLK, 128 - EMB_DIM), jnp.float32)


_xpose_pad = pl.pallas_call(
    _xpose_pad_kernel,
    grid=(_TGRID,),
    in_specs=[pl.BlockSpec((EMB_DIM, _TBLK), lambda i: (0, i))],
    out_specs=pl.BlockSpec((_TBLK, 128), lambda i: (i, 0)),
    out_shape=jax.ShapeDtypeStruct((NUM_CLASSES, 128), jnp.float32),
)


@jax.jit
def kernel(class_indices, table):
    tpad = _xpose_pad(table.T)
    idx = class_indices.reshape(_NW, _NCHUNKS, _CHUNK)
    padded = _gather(idx, tpad)
    return padded[:, :EMB_DIM]


# speedup vs baseline: 2.6039x; 1.3409x over previous
import functools

import jax
import jax.numpy as jnp
from jax import lax
from jax.experimental import pallas as pl
from jax.experimental.pallas import tpu as pltpu, tpu_sc as plsc

NUM_CLASSES = 100000
EMB_DIM = 64
BATCH = 16384

_NW = 32
_B_PER_W = BATCH // _NW
_CHUNK = 128
_NCHUNKS = _B_PER_W // _CHUNK


def _make_gather():
    mesh = plsc.VectorSubcoreMesh(core_axis_name="c", subcore_axis_name="s")

    @functools.partial(
        pl.kernel,
        mesh=mesh,
        out_type=jax.ShapeDtypeStruct((BATCH, 128), jnp.float32),
        scratch_types=[
            pltpu.VMEM((_NCHUNKS, _CHUNK), jnp.int32),
            pltpu.VMEM((_B_PER_W, 128), jnp.float32),
            pltpu.SemaphoreType.DMA((_NCHUNKS,)),
            pltpu.SemaphoreType.DMA,
        ],
    )
    def gather_kernel(idx_hbm, tpad_hbm, out_hbm, idx_v, rows_v, sem, sem_w):
        wid = lax.axis_index("s") * 2 + lax.axis_index("c")
        base = wid * _B_PER_W
        pltpu.sync_copy(idx_hbm.at[wid], idx_v)
        gathers = []
        writes = []
        for j in range(_NCHUNKS):
            gathers.append(
                pltpu.make_async_copy(
                    tpad_hbm.at[idx_v.at[j]],
                    rows_v.at[pl.ds(j * _CHUNK, _CHUNK)],
                    sem.at[j],
                )
            )
            gathers[-1].start()
        for j in range(_NCHUNKS):
            gathers[j].wait()
            writes.append(
                pltpu.make_async_copy(
                    rows_v.at[pl.ds(j * _CHUNK, _CHUNK)],
                    out_hbm.at[pl.ds(base + j * _CHUNK, _CHUNK)],
                    sem_w,
                )
            )
            writes[-1].start()
        for w in writes:
            w.wait()

    return gather_kernel


_gather = _make_gather()

_TBLK = 8192
_TGRID = (NUM_CLASSES + _TBLK - 1) // _TBLK


def _xpose_pad_kernel(t2_ref, out_ref):
    out_ref[:, :EMB_DIM] = t2_ref[...].T
    out_ref[:, EMB_DIM:] = jnp.zeros((_TBLK, 128 - EMB_DIM), jnp.float32)


_xpose_pad = pl.pallas_call(
    _xpose_pad_kernel,
    grid=(_TGRID,),
    in_specs=[pl.BlockSpec((EMB_DIM, _TBLK), lambda i: (0, i))],
    out_specs=pl.BlockSpec((_TBLK, 128), lambda i: (i, 0)),
    out_shape=jax.ShapeDtypeStruct((NUM_CLASSES, 128), jnp.float32),
)


@jax.jit
def kernel(class_indices, table):
    tpad = _xpose_pad(table.T)
    idx = class_indices.reshape(_NW, _NCHUNKS, _CHUNK)
    padded = _gather(idx, tpad)
    return padded[:, :EMB_DIM]


# speedup vs baseline: 2.6759x; 1.0277x over previous
import functools

import jax
import jax.numpy as jnp
from jax import lax
from jax.experimental import pallas as pl
from jax.experimental.pallas import tpu as pltpu, tpu_sc as plsc

NUM_CLASSES = 100000
EMB_DIM = 64
BATCH = 16384

_NW = 32
_B_PER_W = BATCH // _NW
_CHUNK = 128
_NCHUNKS = _B_PER_W // _CHUNK


def _make_gather():
    mesh = plsc.VectorSubcoreMesh(core_axis_name="c", subcore_axis_name="s")

    @functools.partial(
        pl.kernel,
        mesh=mesh,
        out_type=jax.ShapeDtypeStruct((BATCH, 128), jnp.float32),
        scratch_types=[
            pltpu.VMEM((_NCHUNKS, _CHUNK), jnp.int32),
            pltpu.VMEM((_B_PER_W, 128), jnp.float32),
            pltpu.SemaphoreType.DMA((_NCHUNKS,)),
            pltpu.SemaphoreType.DMA,
        ],
    )
    def gather_kernel(idx_hbm, tpad_hbm, out_hbm, idx_v, rows_v, sem, sem_w):
        wid = lax.axis_index("s") * 2 + lax.axis_index("c")
        base = wid * _B_PER_W
        pltpu.sync_copy(idx_hbm.at[wid], idx_v)
        gathers = []
        writes = []
        for j in range(_NCHUNKS):
            gathers.append(
                pltpu.make_async_copy(
                    tpad_hbm.at[idx_v.at[j]],
                    rows_v.at[pl.ds(j * _CHUNK, _CHUNK)],
                    sem.at[j],
                )
            )
            gathers[-1].start()
        for j in range(_NCHUNKS):
            gathers[j].wait()
            writes.append(
                pltpu.make_async_copy(
                    rows_v.at[pl.ds(j * _CHUNK, _CHUNK)],
                    out_hbm.at[pl.ds(base + j * _CHUNK, _CHUNK)],
                    sem_w,
                )
            )
            writes[-1].start()
        for w in writes:
            w.wait()

    return gather_kernel


_gather = _make_gather()

_TBLK = 16384
_TGRID = (NUM_CLASSES + _TBLK - 1) // _TBLK


def _xpose_pad_kernel(t2_ref, out_ref):
    out_ref[:, :EMB_DIM] = t2_ref[...].T
    out_ref[:, EMB_DIM:] = jnp.zeros((_TBLK, 128 - EMB_DIM), jnp.float32)


_xpose_pad = pl.pallas_call(
    _xpose_pad_kernel,
    grid=(_TGRID,),
    in_specs=[pl.BlockSpec((EMB_DIM, _TBLK), lambda i: (0, i))],
    out_specs=pl.BlockSpec((_TBLK, 128), lambda i: (i, 0)),
    out_shape=jax.ShapeDtypeStruct((NUM_CLASSES, 128), jnp.float32),
)


@jax.jit
def kernel(class_indices, table):
    tpad = _xpose_pad(table.T)
    idx = class_indices.reshape(_NW, _NCHUNKS, _CHUNK)
    padded = _gather(idx, tpad)
    return padded[:, :EMB_DIM]
